# Initial kernel scaffold; baseline (speedup 1.0000x reference)
import functools
import math

import jax
import jax.numpy as jnp
from jax import lax
from jax.experimental import pallas as pl
from jax.experimental.pallas import tpu as pltpu
from jax.experimental.pallas import tpu_sc as plsc

NC = 2
NS = 16
NW = NC * NS
CHUNK = 128


def _mesh():
    return plsc.VectorSubcoreMesh(core_axis_name="c", subcore_axis_name="s")


def _deg_sc(dst2, ones_hbm, zeros_hbm, n, k_chunks):
    n_pad = zeros_hbm.shape[0]
    rz = n_pad // NS
    ro = n // NS

    @functools.partial(
        pl.kernel,
        out_type=jax.ShapeDtypeStruct((NC, n, 16), jnp.float32),
        mesh=_mesh(),
        scratch_types=[
            pltpu.VMEM((1, CHUNK), jnp.int32),
            pltpu.VMEM((CHUNK, 16), jnp.float32),
            pltpu.VMEM_SHARED((n_pad, 16), jnp.float32),
            pltpu.SemaphoreType.DMA,
        ],
    )
    def deg_kernel(dst_hbm, ones_h, zeros_h, out_hbm, idx_v, ones_v, acc, sem):
        cid = lax.axis_index("c")
        sid = lax.axis_index("s")
        wid = cid * NS + sid
        pltpu.sync_copy(zeros_h.at[pl.ds(sid * rz, rz)], acc.at[pl.ds(sid * rz, rz)])
        pltpu.sync_copy(ones_h, ones_v)
        plsc.subcore_barrier()

        @pl.loop(0, k_chunks)
        def _(k):
            row = wid * k_chunks + k
            pltpu.sync_copy(dst_hbm.at[pl.ds(row, 1)], idx_v)
            pltpu.sync_copy(ones_v, acc.at[idx_v.at[0]], add=True)

        plsc.subcore_barrier()
        pltpu.sync_copy(acc.at[pl.ds(sid * ro, ro)],
                        out_hbm.at[cid].at[pl.ds(sid * ro, ro)])

    return deg_kernel(dst2, ones_hbm, zeros_hbm)


def _agg_sc(hs_pad, src2, dst2, zeros_hbm, n, d, k_chunks):
    n_pad = zeros_hbm.shape[0]
    rz = n_pad // NS
    ro = n // NS

    @functools.partial(
        pl.kernel,
        out_type=jax.ShapeDtypeStruct((NC, n, d), jnp.float32),
        mesh=_mesh(),
        scratch_types=[
            pltpu.VMEM((1, CHUNK), jnp.int32),
            pltpu.VMEM((1, CHUNK), jnp.int32),
            pltpu.VMEM((CHUNK, d), jnp.float32),
            pltpu.VMEM_SHARED((n_pad, d), jnp.float32),
            pltpu.SemaphoreType.DMA,
        ],
    )
    def agg_kernel(hs_hbm, src_hbm, dst_hbm, zeros_h, out_hbm,
                   sidx, didx, rows_v, acc, sem):
        cid = lax.axis_index("c")
        sid = lax.axis_index("s")
        wid = cid * NS + sid
        pltpu.sync_copy(zeros_h.at[pl.ds(sid * rz, rz)], acc.at[pl.ds(sid * rz, rz)])
        plsc.subcore_barrier()

        @pl.loop(0, k_chunks)
        def _(k):
            row = wid * k_chunks + k
            pltpu.sync_copy(src_hbm.at[pl.ds(row, 1)], sidx)
            pltpu.sync_copy(dst_hbm.at[pl.ds(row, 1)], didx)
            pltpu.async_copy(hs_hbm.at[sidx.at[0]], rows_v, sem).wait()
            pltpu.sync_copy(rows_v, acc.at[didx.at[0]], add=True)

        plsc.subcore_barrier()
        pltpu.sync_copy(acc.at[pl.ds(sid * ro, ro)],
                        out_hbm.at[cid].at[pl.ds(sid * ro, ro)])

    return agg_kernel(hs_pad, src2, dst2, zeros_hbm)


def _matmul_tc(x, W):
    n, d_in = x.shape
    d_out = W.shape[0]

    def mm(x_ref, w_ref, o_ref):
        o_ref[...] = lax.dot_general(
            x_ref[...], w_ref[...],
            dimension_numbers=(((1,), (1,)), ((), ())),
            preferred_element_type=jnp.float32)

    return pl.pallas_call(
        mm, out_shape=jax.ShapeDtypeStruct((n, d_out), jnp.float32))(x, W)


def _scale_tc(h, degp, n_rows_pad):
    n, d = h.shape

    def scale(h_ref, degp_ref, o_ref):
        deg = degp_ref[0, :, 0:1] + degp_ref[1, :, 0:1] + 1.0
        o_ref[0:n, :] = h_ref[...] * lax.rsqrt(deg)
        o_ref[n:n_rows_pad, :] = jnp.zeros((n_rows_pad - n, d), jnp.float32)

    return pl.pallas_call(
        scale, out_shape=jax.ShapeDtypeStruct((n_rows_pad, d), jnp.float32))(h, degp)


def _final_tc(accp, hs_pad, degp, x, b, gamma, beta):
    n, d = x.shape
    bn_scale = float(1.0 / math.sqrt(1.0 + 1e-5))

    def final(accp_ref, hs_ref, degp_ref, x_ref, b_ref, g_ref, be_ref, o_ref):
        s = accp_ref[0] + accp_ref[1]
        deg = degp_ref[0, :, 0:1] + degp_ref[1, :, 0:1] + 1.0
        dinv = lax.rsqrt(deg)
        agg = (s + hs_ref[0:n, :]) * dinv
        y = (agg + b_ref[...]) * bn_scale * g_ref[...] + be_ref[...]
        o_ref[...] = jnp.maximum(y, 0.0) + x_ref[...]

    return pl.pallas_call(
        final, out_shape=jax.ShapeDtypeStruct((n, d), jnp.float32))(
            accp, hs_pad, degp, x,
            b.reshape(1, d), gamma.reshape(1, d), beta.reshape(1, d))


def kernel(x, edge_index, W, b, gamma, beta):
    n, d_in = x.shape
    d_out = W.shape[0]
    e = edge_index.shape[1]

    k_chunks = -(-e // (NW * CHUNK))
    e_pad = NW * k_chunks * CHUNK
    pad = e_pad - e
    src = jnp.concatenate([edge_index[0], jnp.full((pad,), n, jnp.int32)])
    dst = jnp.concatenate([edge_index[1], jnp.full((pad,), n, jnp.int32)])
    src2 = src.reshape(NW * k_chunks, CHUNK)
    dst2 = dst.reshape(NW * k_chunks, CHUNK)

    n_pad = n + NS
    zeros16 = jnp.zeros((n_pad, 16), jnp.float32)
    zeros_d = jnp.zeros((n_pad, d_out), jnp.float32)
    ones16 = jnp.ones((CHUNK, 16), jnp.float32)

    degp = _deg_sc(dst2, ones16, zeros16, n, k_chunks)
    h = _matmul_tc(x, W)
    hs_pad = _scale_tc(h, degp, n + 8)
    accp = _agg_sc(hs_pad, src2, dst2, zeros_d, n, d_out, k_chunks)
    return _final_tc(accp, hs_pad, degp, x, b, gamma, beta)


# speedup vs baseline: 12.2937x; 12.2937x over previous
import functools
import math

import jax
import jax.numpy as jnp
from jax import lax
from jax.experimental import pallas as pl
from jax.experimental.pallas import tpu as pltpu
from jax.experimental.pallas import tpu_sc as plsc

NC = 2
NS = 16
NW = NC * NS
CHUNK = 128


def _mesh():
    return plsc.VectorSubcoreMesh(core_axis_name="c", subcore_axis_name="s")


def _deg_sc(dst2, ones_hbm, zeros_hbm, n, k_chunks):
    n_pad = zeros_hbm.shape[0]
    rz = n_pad // NS

    @functools.partial(
        pl.kernel,
        out_type=jax.ShapeDtypeStruct((NC, n_pad, 16), jnp.float32),
        mesh=_mesh(),
        scratch_types=[
            pltpu.VMEM((1, CHUNK), jnp.int32),
            pltpu.VMEM((CHUNK, 16), jnp.float32),
            pltpu.VMEM_SHARED((n_pad, 16), jnp.float32),
            pltpu.SemaphoreType.DMA,
        ],
    )
    def deg_kernel(dst_hbm, ones_h, zeros_h, out_hbm, idx_v, ones_v, acc, sem):
        cid = lax.axis_index("c")
        sid = lax.axis_index("s")
        wid = cid * NS + sid
        pltpu.sync_copy(zeros_h.at[pl.ds(sid * rz, rz)], acc.at[pl.ds(sid * rz, rz)])
        pltpu.sync_copy(ones_h, ones_v)
        plsc.subcore_barrier()

        @pl.loop(0, k_chunks)
        def _(k):
            row = wid * k_chunks + k
            pltpu.sync_copy(dst_hbm.at[pl.ds(row, 1)], idx_v)
            pltpu.sync_copy(ones_v, acc.at[idx_v.at[0]], add=True)

        plsc.subcore_barrier()
        pltpu.sync_copy(acc.at[pl.ds(sid * rz, rz)],
                        out_hbm.at[cid].at[pl.ds(sid * rz, rz)])

    return deg_kernel(dst2, ones_hbm, zeros_hbm)


def _agg_sc(hs_pad, src2, dst2, zeros_hbm, n, d, k_chunks):
    n_pad = zeros_hbm.shape[0]
    rz = n_pad // NS

    @functools.partial(
        pl.kernel,
        out_type=jax.ShapeDtypeStruct((NC, n_pad, d), jnp.float32),
        mesh=_mesh(),
        scratch_types=[
            pltpu.VMEM((1, CHUNK), jnp.int32),
            pltpu.VMEM((1, CHUNK), jnp.int32),
            pltpu.VMEM((CHUNK, d), jnp.float32),
            pltpu.VMEM_SHARED((n_pad, d), jnp.float32),
            pltpu.SemaphoreType.DMA,
        ],
    )
    def agg_kernel(hs_hbm, src_hbm, dst_hbm, zeros_h, out_hbm,
                   sidx, didx, rows_v, acc, sem):
        cid = lax.axis_index("c")
        sid = lax.axis_index("s")
        wid = cid * NS + sid
        pltpu.sync_copy(zeros_h.at[pl.ds(sid * rz, rz)], acc.at[pl.ds(sid * rz, rz)])
        plsc.subcore_barrier()

        @pl.loop(0, k_chunks)
        def _(k):
            row = wid * k_chunks + k
            pltpu.sync_copy(src_hbm.at[pl.ds(row, 1)], sidx)
            pltpu.sync_copy(dst_hbm.at[pl.ds(row, 1)], didx)
            pltpu.async_copy(hs_hbm.at[sidx.at[0]], rows_v, sem).wait()
            pltpu.sync_copy(rows_v, acc.at[didx.at[0]], add=True)

        plsc.subcore_barrier()
        pltpu.sync_copy(acc.at[pl.ds(sid * rz, rz)],
                        out_hbm.at[cid].at[pl.ds(sid * rz, rz)])

    return agg_kernel(hs_pad, src2, dst2, zeros_hbm)


def _matmul_tc(x, W):
    n, d_in = x.shape
    d_out = W.shape[0]

    def mm(x_ref, w_ref, o_ref):
        o_ref[...] = lax.dot_general(
            x_ref[...], w_ref[...],
            dimension_numbers=(((1,), (1,)), ((), ())),
            preferred_element_type=jnp.float32)

    return pl.pallas_call(
        mm, out_shape=jax.ShapeDtypeStruct((n, d_out), jnp.float32))(x, W)


def _scale_tc(h, degp, n_rows_pad):
    n, d = h.shape

    def scale(h_ref, degp_ref, o_ref):
        deg = degp_ref[0, 0:n, 0:1] + degp_ref[1, 0:n, 0:1] + 1.0
        o_ref[0:n, :] = h_ref[...] * lax.rsqrt(deg)
        o_ref[n:n_rows_pad, :] = jnp.zeros((n_rows_pad - n, d), jnp.float32)

    return pl.pallas_call(
        scale, out_shape=jax.ShapeDtypeStruct((n_rows_pad, d), jnp.float32))(h, degp)


def _final_tc(accp, hs_pad, degp, x, b, gamma, beta):
    n, d = x.shape
    bn_scale = float(1.0 / math.sqrt(1.0 + 1e-5))

    def final(accp_ref, hs_ref, degp_ref, x_ref, b_ref, g_ref, be_ref, o_ref):
        s = accp_ref[0, 0:n, :] + accp_ref[1, 0:n, :]
        deg = degp_ref[0, 0:n, 0:1] + degp_ref[1, 0:n, 0:1] + 1.0
        dinv = lax.rsqrt(deg)
        agg = (s + hs_ref[0:n, :]) * dinv
        y = (agg + b_ref[...]) * bn_scale * g_ref[...] + be_ref[...]
        o_ref[...] = jnp.maximum(y, 0.0) + x_ref[...]

    return pl.pallas_call(
        final, out_shape=jax.ShapeDtypeStruct((n, d), jnp.float32))(
            accp, hs_pad, degp, x,
            b.reshape(1, d), gamma.reshape(1, d), beta.reshape(1, d))


def kernel(x, edge_index, W, b, gamma, beta):
    n, d_in = x.shape
    d_out = W.shape[0]
    e = edge_index.shape[1]

    k_chunks = -(-e // (NW * CHUNK))
    e_pad = NW * k_chunks * CHUNK
    pad = e_pad - e
    src = jnp.concatenate([edge_index[0], jnp.full((pad,), n, jnp.int32)])
    dst = jnp.concatenate([edge_index[1], jnp.full((pad,), n, jnp.int32)])
    src2 = src.reshape(NW * k_chunks, CHUNK)
    dst2 = dst.reshape(NW * k_chunks, CHUNK)

    n_pad = -(-n // (8 * NS)) * (8 * NS)
    zeros16 = jnp.zeros((n_pad, 16), jnp.float32)
    zeros_d = jnp.zeros((n_pad, d_out), jnp.float32)
    ones16 = jnp.ones((CHUNK, 16), jnp.float32)

    degp = _deg_sc(dst2, ones16, zeros16, n, k_chunks)
    h = _matmul_tc(x, W)
    hs_pad = _scale_tc(h, degp, n + 8)
    accp = _agg_sc(hs_pad, src2, dst2, zeros_d, n, d_out, k_chunks)
    return _final_tc(accp, hs_pad, degp, x, b, gamma, beta)
